# baseline (speedup 1.0000x reference)
import functools

import jax
import jax.numpy as jnp
from jax import lax
from jax.experimental import pallas as pl
from jax.experimental.pallas import tpu as pltpu
from jax.experimental.pallas import tpu_sc as plsc

N = 25000
NPAD = 25600
E = 800000
EPAD = 819200
H = 64
O = 32
EPT = EPAD // 16
CHA = 6400
CHC = 8192
GRP = 16
BLK = 3200
ON = O * NPAD

_mesh = plsc.VectorSubcoreMesh(core_axis_name="c", subcore_axis_name="s")
_sc_params = pltpu.CompilerParams(needs_layout_passes=False)


def _edge_loop(pk_hbm, pk_v, sem, pairs, base, nch, ch_size, grp=GRP):
    pltpu.async_copy(pk_hbm.at[pl.ds(base, ch_size)],
                     pk_v.at[pl.ds(0, ch_size)], sem)

    def chunk(ch, carry):
        boff = (ch % 2) * ch_size
        pltpu.make_async_copy(pk_hbm.at[pl.ds(base, ch_size)],
                              pk_v.at[pl.ds(boff, ch_size)], sem).wait()

        @pl.when(ch + 1 < nch)
        def _issue():
            nboff = ((ch + 1) % 2) * ch_size
            pltpu.async_copy(pk_hbm.at[pl.ds(base + (ch + 1) * ch_size, ch_size)],
                             pk_v.at[pl.ds(nboff, ch_size)], sem)

        def vstep(v, c2):
            b = boff + v * (16 * grp)
            ps = [pk_v[pl.ds(b + 16 * k, 16)] for k in range(grp)]
            ss = [jnp.bitwise_and(p, 0xFFFF) for p in ps]
            dd = [jnp.right_shift(p, 16) for p in ps]
            vals = [[plsc.load_gather(tab_v, [s]) for tab_v, _ in pairs]
                    for s in ss]
            for vs, d in zip(vals, dd):
                for (_, acc_v), val in zip(pairs, vs):
                    plsc.addupdate_scatter(acc_v, [d], val)
            return c2

        lax.fori_loop(0, ch_size // (16 * grp), vstep, 0, unroll=1)
        return carry

    lax.fori_loop(0, nch, chunk, 0)


@functools.partial(
    pl.kernel,
    out_type=jax.ShapeDtypeStruct((2 * 16 * NPAD,), jnp.float32),
    mesh=_mesh,
    compiler_params=_sc_params,
    scratch_types=[
        pltpu.VMEM((NPAD,), jnp.float32),
        pltpu.VMEM((NPAD,), jnp.float32),
        pltpu.VMEM((2 * CHA,), jnp.int32),
        pltpu.SemaphoreType.DMA,
    ],
)
def _sc_layer1(x2_hbm, pk2_hbm, aggp_hbm, tab_v, acc_v, pk_v, sem):
    core = lax.axis_index("c")
    s = lax.axis_index("s")
    pltpu.sync_copy(x2_hbm.at[pl.ds(core * NPAD, NPAD)], tab_v)

    def zero(i, carry):
        acc_v[pl.ds(i * 16, 16)] = jnp.zeros((16,), jnp.float32)
        return carry

    lax.fori_loop(0, NPAD // 16, zero, 0, unroll=8)
    _edge_loop(pk2_hbm, pk_v, sem, [(tab_v, acc_v)],
               core * EPAD + s * EPT, EPT // CHA, CHA)
    pltpu.sync_copy(acc_v, aggp_hbm.at[pl.ds((core * 16 + s) * NPAD, NPAD)])


@functools.partial(
    pl.kernel,
    out_type=jax.ShapeDtypeStruct((2 * ON,), jnp.float32),
    mesh=_mesh,
    compiler_params=_sc_params,
    scratch_types=[
        pltpu.VMEM((NPAD,), jnp.float32),
        pltpu.VMEM((NPAD,), jnp.float32),
        pltpu.VMEM((NPAD,), jnp.float32),
        pltpu.VMEM((NPAD,), jnp.float32),
        pltpu.VMEM((2 * CHC,), jnp.int32),
        pltpu.SemaphoreType.DMA,
    ],
)
def _sc_layer2(z2_hbm, pk2_hbm, rb2_hbm, out2_hbm,
               tab0_v, acc0_v, tab1_v, acc1_v, pk_v, sem):
    core = lax.axis_index("c")
    c0 = lax.axis_index("s") * 2
    zb = core * ON + c0 * NPAD
    rbb = (1 - core) * ON + c0 * NPAD
    ob = core * ON + c0 * NPAD
    pltpu.sync_copy(z2_hbm.at[pl.ds(zb, NPAD)], tab0_v)
    pltpu.sync_copy(z2_hbm.at[pl.ds(zb + NPAD, NPAD)], tab1_v)
    pltpu.sync_copy(rb2_hbm.at[pl.ds(rbb, NPAD)], acc0_v)
    pltpu.sync_copy(rb2_hbm.at[pl.ds(rbb + NPAD, NPAD)], acc1_v)
    _edge_loop(pk2_hbm, pk_v, sem, [(tab0_v, acc0_v), (tab1_v, acc1_v)],
               core * EPAD, EPAD // CHC, CHC, grp=32)
    pltpu.sync_copy(acc0_v, out2_hbm.at[pl.ds(ob, NPAD)])
    pltpu.sync_copy(acc1_v, out2_hbm.at[pl.ds(ob + NPAD, NPAD)])


def _dense_body(aggp_ref, x_ref, w3_ref, w2relT_ref, w2rootT_ref, b2_ref,
                z_ref, rb_ref):
    agg = jnp.sum(aggp_ref[0], axis=0, keepdims=True)
    x = x_ref[0]
    ones = jnp.ones((1, x.shape[1]), jnp.float32)
    zeros = jnp.zeros((5, x.shape[1]), jnp.float32)
    a3 = jnp.concatenate([agg, x, ones, zeros], axis=0)
    hT = jax.nn.relu(
        jnp.dot(w3_ref[0], a3, preferred_element_type=jnp.float32))
    z_ref[...] = jnp.dot(w2relT_ref[0], hT,
                         preferred_element_type=jnp.float32)[None]
    rb_ref[...] = (jnp.dot(w2rootT_ref[0], hT,
                           preferred_element_type=jnp.float32)
                   + b2_ref[0][:, 0:1])[None]


def _dense(aggp, x2d, w3s, w2relTs, w2rootTs, b2ws):
    nb = NPAD // BLK
    return pl.pallas_call(
        _dense_body,
        grid=(2, nb),
        in_specs=[
            pl.BlockSpec((1, 16, BLK), lambda d, i: (1 - d, 0, i)),
            pl.BlockSpec((1, 1, BLK), lambda d, i: (d, 0, i)),
            pl.BlockSpec((1, H, 8), lambda d, i: (d, 0, 0)),
            pl.BlockSpec((1, O, H), lambda d, i: (d, 0, 0)),
            pl.BlockSpec((1, O, H), lambda d, i: (d, 0, 0)),
            pl.BlockSpec((1, O, 128), lambda d, i: (d, 0, 0)),
        ],
        out_specs=[pl.BlockSpec((1, O, BLK), lambda d, i: (d, 0, i)),
                   pl.BlockSpec((1, O, BLK), lambda d, i: (d, 0, i))],
        out_shape=[jax.ShapeDtypeStruct((2, O, NPAD), jnp.float32),
                   jax.ShapeDtypeStruct((2, O, NPAD), jnp.float32)],
    )(aggp, x2d, w3s, w2relTs, w2rootTs, b2ws)


def _pack(edge_index):
    pk = jnp.bitwise_or(edge_index[0],
                        jnp.left_shift(edge_index[1], 16)).astype(jnp.int32)
    pad = jnp.full((EPAD - E,), N | (N << 16), dtype=jnp.int32)
    return jnp.concatenate([pk, pad])


def _w3(w_rel, w_root, b):
    w = jnp.concatenate([w_rel, w_root, b[None, :],
                         jnp.zeros((5, H), jnp.float32)], axis=0)
    return w.T


def kernel(x_user, x_item, edge_index_u2i, edge_index_i2u,
           W1_rel_u2i, b1_rel_u2i, W1_root_u2i,
           W1_rel_i2u, b1_rel_i2u, W1_root_i2u,
           W2_rel_u2i, b2_rel_u2i, W2_root_u2i,
           W2_rel_i2u, b2_rel_i2u, W2_root_i2u):
    xu = jnp.pad(x_user[:, 0], (0, NPAD - N))
    xi = jnp.pad(x_item[:, 0], (0, NPAD - N))
    x2 = jnp.concatenate([xu, xi])
    pk2 = jnp.concatenate([_pack(edge_index_u2i), _pack(edge_index_i2u)])

    aggp = _sc_layer1(x2, pk2).reshape(2, 16, NPAD)

    x2d = x2.reshape(2, 1, NPAD)
    w3s = jnp.stack([_w3(W1_rel_i2u, W1_root_i2u, b1_rel_i2u),
                     _w3(W1_rel_u2i, W1_root_u2i, b1_rel_u2i)])
    w2relTs = jnp.stack([W2_rel_u2i.T, W2_rel_i2u.T])
    w2rootTs = jnp.stack([W2_root_i2u.T, W2_root_u2i.T])
    b2ws = jnp.stack([jnp.broadcast_to(b2_rel_i2u[:, None], (O, 128)),
                      jnp.broadcast_to(b2_rel_u2i[:, None], (O, 128))])
    zs, rbs = _dense(aggp, x2d, w3s, w2relTs, w2rootTs, b2ws)

    out2 = _sc_layer2(zs.reshape(-1), pk2, rbs.reshape(-1))
    out2 = out2.reshape(2, O, NPAD)
    out_item = out2[0].T[:N]
    out_user = out2[1].T[:N]
    return (out_user, out_item)


# speedup vs baseline: 1.3953x; 1.3953x over previous
import functools

import jax
import jax.numpy as jnp
from jax import lax
from jax.experimental import pallas as pl
from jax.experimental.pallas import tpu as pltpu
from jax.experimental.pallas import tpu_sc as plsc

N = 25000
NPAD = 25600
E = 800000
EPAD = 819200
H = 64
O = 32
EPT = EPAD // 16
CHA = 6400
CHC = 12800
GRP = 16
BLK = 3200
ON = O * NPAD

_mesh = plsc.VectorSubcoreMesh(core_axis_name="c", subcore_axis_name="s")
_sc_params = pltpu.CompilerParams(needs_layout_passes=False)


def _edge_loop(pk_hbm, pk_v, sem, pairs, base, nch, ch_size, grp=GRP,
               unroll=1):
    pltpu.async_copy(pk_hbm.at[pl.ds(base, ch_size)],
                     pk_v.at[pl.ds(0, ch_size)], sem)

    def chunk(ch, carry):
        boff = (ch % 2) * ch_size
        pltpu.make_async_copy(pk_hbm.at[pl.ds(base, ch_size)],
                              pk_v.at[pl.ds(boff, ch_size)], sem).wait()

        @pl.when(ch + 1 < nch)
        def _issue():
            nboff = ((ch + 1) % 2) * ch_size
            pltpu.async_copy(pk_hbm.at[pl.ds(base + (ch + 1) * ch_size, ch_size)],
                             pk_v.at[pl.ds(nboff, ch_size)], sem)

        def vstep(v, c2):
            b = boff + v * (16 * grp)
            ps = [pk_v[pl.ds(b + 16 * k, 16)] for k in range(grp)]
            ss = [jnp.bitwise_and(p, 0xFFFF) for p in ps]
            dd = [jnp.right_shift(p, 16) for p in ps]
            vals = [[plsc.load_gather(tab_v, [s]) for tab_v, _ in pairs]
                    for s in ss]
            for vs, d in zip(vals, dd):
                for (_, acc_v), val in zip(pairs, vs):
                    plsc.addupdate_scatter(acc_v, [d], val)
            return c2

        lax.fori_loop(0, ch_size // (16 * grp), vstep, 0, unroll=unroll)
        return carry

    lax.fori_loop(0, nch, chunk, 0)


@functools.partial(
    pl.kernel,
    out_type=jax.ShapeDtypeStruct((2 * 16 * NPAD,), jnp.float32),
    mesh=_mesh,
    compiler_params=_sc_params,
    scratch_types=[
        pltpu.VMEM((NPAD,), jnp.float32),
        pltpu.VMEM((NPAD,), jnp.float32),
        pltpu.VMEM((2 * CHA,), jnp.int32),
        pltpu.SemaphoreType.DMA,
    ],
)
def _sc_layer1(x2_hbm, pk2_hbm, aggp_hbm, tab_v, acc_v, pk_v, sem):
    core = lax.axis_index("c")
    s = lax.axis_index("s")
    pltpu.sync_copy(x2_hbm.at[pl.ds(core * NPAD, NPAD)], tab_v)

    def zero(i, carry):
        acc_v[pl.ds(i * 16, 16)] = jnp.zeros((16,), jnp.float32)
        return carry

    lax.fori_loop(0, NPAD // 16, zero, 0, unroll=8)
    _edge_loop(pk2_hbm, pk_v, sem, [(tab_v, acc_v)],
               core * EPAD + s * EPT, EPT // CHA, CHA)
    pltpu.sync_copy(acc_v, aggp_hbm.at[pl.ds((core * 16 + s) * NPAD, NPAD)])


@functools.partial(
    pl.kernel,
    out_type=jax.ShapeDtypeStruct((2 * ON,), jnp.float32),
    mesh=_mesh,
    compiler_params=_sc_params,
    scratch_types=[
        pltpu.VMEM((NPAD,), jnp.float32),
        pltpu.VMEM((NPAD,), jnp.float32),
        pltpu.VMEM((NPAD,), jnp.float32),
        pltpu.VMEM((NPAD,), jnp.float32),
        pltpu.VMEM((2 * CHC,), jnp.int32),
        pltpu.SemaphoreType.DMA,
    ],
)
def _sc_layer2(z2_hbm, pk2_hbm, rb2_hbm, out2_hbm,
               tab0_v, acc0_v, tab1_v, acc1_v, pk_v, sem):
    core = lax.axis_index("c")
    c0 = lax.axis_index("s") * 2
    zb = core * ON + c0 * NPAD
    rbb = (1 - core) * ON + c0 * NPAD
    ob = core * ON + c0 * NPAD
    pltpu.sync_copy(z2_hbm.at[pl.ds(zb, NPAD)], tab0_v)
    pltpu.sync_copy(z2_hbm.at[pl.ds(zb + NPAD, NPAD)], tab1_v)
    pltpu.sync_copy(rb2_hbm.at[pl.ds(rbb, NPAD)], acc0_v)
    pltpu.sync_copy(rb2_hbm.at[pl.ds(rbb + NPAD, NPAD)], acc1_v)
    _edge_loop(pk2_hbm, pk_v, sem, [(tab0_v, acc0_v), (tab1_v, acc1_v)],
               core * EPAD, EPAD // CHC, CHC, unroll=2)
    pltpu.sync_copy(acc0_v, out2_hbm.at[pl.ds(ob, NPAD)])
    pltpu.sync_copy(acc1_v, out2_hbm.at[pl.ds(ob + NPAD, NPAD)])


def _dense_body(aggp_ref, x_ref, w3_ref, w2relT_ref, w2rootT_ref, b2_ref,
                z_ref, rb_ref):
    agg = jnp.sum(aggp_ref[0], axis=0, keepdims=True)
    x = x_ref[0]
    ones = jnp.ones((1, x.shape[1]), jnp.float32)
    zeros = jnp.zeros((5, x.shape[1]), jnp.float32)
    a3 = jnp.concatenate([agg, x, ones, zeros], axis=0)
    hT = jax.nn.relu(
        jnp.dot(w3_ref[0], a3, preferred_element_type=jnp.float32))
    z_ref[...] = jnp.dot(w2relT_ref[0], hT,
                         preferred_element_type=jnp.float32)[None]
    rb_ref[...] = (jnp.dot(w2rootT_ref[0], hT,
                           preferred_element_type=jnp.float32)
                   + b2_ref[0][:, 0:1])[None]


def _dense(aggp, x2d, w3s, w2relTs, w2rootTs, b2ws):
    nb = NPAD // BLK
    return pl.pallas_call(
        _dense_body,
        grid=(2, nb),
        in_specs=[
            pl.BlockSpec((1, 16, BLK), lambda d, i: (1 - d, 0, i)),
            pl.BlockSpec((1, 1, BLK), lambda d, i: (d, 0, i)),
            pl.BlockSpec((1, H, 8), lambda d, i: (d, 0, 0)),
            pl.BlockSpec((1, O, H), lambda d, i: (d, 0, 0)),
            pl.BlockSpec((1, O, H), lambda d, i: (d, 0, 0)),
            pl.BlockSpec((1, O, 128), lambda d, i: (d, 0, 0)),
        ],
        out_specs=[pl.BlockSpec((1, O, BLK), lambda d, i: (d, 0, i)),
                   pl.BlockSpec((1, O, BLK), lambda d, i: (d, 0, i))],
        out_shape=[jax.ShapeDtypeStruct((2, O, NPAD), jnp.float32),
                   jax.ShapeDtypeStruct((2, O, NPAD), jnp.float32)],
    )(aggp, x2d, w3s, w2relTs, w2rootTs, b2ws)


def _pack(edge_index):
    pk = jnp.bitwise_or(edge_index[0],
                        jnp.left_shift(edge_index[1], 16)).astype(jnp.int32)
    pad = jnp.full((EPAD - E,), N | (N << 16), dtype=jnp.int32)
    return jnp.concatenate([pk, pad])


def _w3(w_rel, w_root, b):
    w = jnp.concatenate([w_rel, w_root, b[None, :],
                         jnp.zeros((5, H), jnp.float32)], axis=0)
    return w.T


def kernel(x_user, x_item, edge_index_u2i, edge_index_i2u,
           W1_rel_u2i, b1_rel_u2i, W1_root_u2i,
           W1_rel_i2u, b1_rel_i2u, W1_root_i2u,
           W2_rel_u2i, b2_rel_u2i, W2_root_u2i,
           W2_rel_i2u, b2_rel_i2u, W2_root_i2u):
    xu = jnp.pad(x_user[:, 0], (0, NPAD - N))
    xi = jnp.pad(x_item[:, 0], (0, NPAD - N))
    x2 = jnp.concatenate([xu, xi])
    pk2 = jnp.concatenate([_pack(edge_index_u2i), _pack(edge_index_i2u)])

    aggp = _sc_layer1(x2, pk2).reshape(2, 16, NPAD)

    x2d = x2.reshape(2, 1, NPAD)
    w3s = jnp.stack([_w3(W1_rel_i2u, W1_root_i2u, b1_rel_i2u),
                     _w3(W1_rel_u2i, W1_root_u2i, b1_rel_u2i)])
    w2relTs = jnp.stack([W2_rel_u2i.T, W2_rel_i2u.T])
    w2rootTs = jnp.stack([W2_root_i2u.T, W2_root_u2i.T])
    b2ws = jnp.stack([jnp.broadcast_to(b2_rel_i2u[:, None], (O, 128)),
                      jnp.broadcast_to(b2_rel_u2i[:, None], (O, 128))])
    zs, rbs = _dense(aggp, x2d, w3s, w2relTs, w2rootTs, b2ws)

    out2 = _sc_layer2(zs.reshape(-1), pk2, rbs.reshape(-1))
    out2 = out2.reshape(2, O, NPAD)
    out_item = out2[0].T[:N]
    out_user = out2[1].T[:N]
    return (out_user, out_item)


# speedup vs baseline: 1.3993x; 1.0029x over previous
import functools

import jax
import jax.numpy as jnp
from jax import lax
from jax.experimental import pallas as pl
from jax.experimental.pallas import tpu as pltpu
from jax.experimental.pallas import tpu_sc as plsc

N = 25000
NPAD = 25600
E = 800000
EPAD = 819200
H = 64
O = 32
EPT = EPAD // 16
CHA = 6400
CHC = 8192
GRP = 16
BLK = 3200
ON = O * NPAD

_mesh = plsc.VectorSubcoreMesh(core_axis_name="c", subcore_axis_name="s")
_sc_params = pltpu.CompilerParams(needs_layout_passes=False)


def _edge_loop(pk_hbm, pk_v, sem, pairs, base, nch, ch_size, grp=GRP,
               unroll=1):
    pltpu.async_copy(pk_hbm.at[pl.ds(base, ch_size)],
                     pk_v.at[pl.ds(0, ch_size)], sem)

    def chunk(ch, carry):
        boff = (ch % 2) * ch_size
        pltpu.make_async_copy(pk_hbm.at[pl.ds(base, ch_size)],
                              pk_v.at[pl.ds(boff, ch_size)], sem).wait()

        @pl.when(ch + 1 < nch)
        def _issue():
            nboff = ((ch + 1) % 2) * ch_size
            pltpu.async_copy(pk_hbm.at[pl.ds(base + (ch + 1) * ch_size, ch_size)],
                             pk_v.at[pl.ds(nboff, ch_size)], sem)

        def vstep(v, c2):
            b = boff + v * (16 * grp)
            ps = [pk_v[pl.ds(b + 16 * k, 16)] for k in range(grp)]
            ss = [jnp.bitwise_and(p, 0xFFFF) for p in ps]
            dd = [jnp.right_shift(p, 16) for p in ps]
            vals = [[plsc.load_gather(tab_v, [s]) for tab_v, _ in pairs]
                    for s in ss]
            for vs, d in zip(vals, dd):
                for (_, acc_v), val in zip(pairs, vs):
                    plsc.addupdate_scatter(acc_v, [d], val)
            return c2

        lax.fori_loop(0, ch_size // (16 * grp), vstep, 0, unroll=unroll)
        return carry

    lax.fori_loop(0, nch, chunk, 0)


@functools.partial(
    pl.kernel,
    out_type=jax.ShapeDtypeStruct((2 * 16 * NPAD,), jnp.float32),
    mesh=_mesh,
    compiler_params=_sc_params,
    scratch_types=[
        pltpu.VMEM((NPAD,), jnp.float32),
        pltpu.VMEM((NPAD,), jnp.float32),
        pltpu.VMEM((2 * CHA,), jnp.int32),
        pltpu.SemaphoreType.DMA,
    ],
)
def _sc_layer1(x2_hbm, pk2_hbm, aggp_hbm, tab_v, acc_v, pk_v, sem):
    core = lax.axis_index("c")
    s = lax.axis_index("s")
    pltpu.sync_copy(x2_hbm.at[pl.ds(core * NPAD, NPAD)], tab_v)

    def zero(i, carry):
        acc_v[pl.ds(i * 16, 16)] = jnp.zeros((16,), jnp.float32)
        return carry

    lax.fori_loop(0, NPAD // 16, zero, 0, unroll=8)
    _edge_loop(pk2_hbm, pk_v, sem, [(tab_v, acc_v)],
               core * EPAD + s * EPT, EPT // CHA, CHA)
    pltpu.sync_copy(acc_v, aggp_hbm.at[pl.ds((core * 16 + s) * NPAD, NPAD)])


@functools.partial(
    pl.kernel,
    out_type=jax.ShapeDtypeStruct((2 * ON,), jnp.float32),
    mesh=_mesh,
    compiler_params=_sc_params,
    scratch_types=[
        pltpu.VMEM((NPAD,), jnp.float32),
        pltpu.VMEM((NPAD,), jnp.float32),
        pltpu.VMEM((NPAD,), jnp.float32),
        pltpu.VMEM((NPAD,), jnp.float32),
        pltpu.VMEM((2 * CHC,), jnp.int32),
        pltpu.SemaphoreType.DMA,
    ],
)
def _sc_layer2(z2_hbm, pk2_hbm, rb2_hbm, out2_hbm,
               tab0_v, acc0_v, tab1_v, acc1_v, pk_v, sem):
    core = lax.axis_index("c")
    c0 = lax.axis_index("s") * 2
    zb = core * ON + c0 * NPAD
    rbb = (1 - core) * ON + c0 * NPAD
    ob = core * ON + c0 * NPAD
    pltpu.sync_copy(z2_hbm.at[pl.ds(zb, NPAD)], tab0_v)
    pltpu.sync_copy(z2_hbm.at[pl.ds(zb + NPAD, NPAD)], tab1_v)
    pltpu.sync_copy(rb2_hbm.at[pl.ds(rbb, NPAD)], acc0_v)
    pltpu.sync_copy(rb2_hbm.at[pl.ds(rbb + NPAD, NPAD)], acc1_v)
    _edge_loop(pk2_hbm, pk_v, sem, [(tab0_v, acc0_v), (tab1_v, acc1_v)],
               core * EPAD, EPAD // CHC, CHC)
    pltpu.sync_copy(acc0_v, out2_hbm.at[pl.ds(ob, NPAD)])
    pltpu.sync_copy(acc1_v, out2_hbm.at[pl.ds(ob + NPAD, NPAD)])


def _dense_body(aggp_ref, x_ref, w3_ref, w2relT_ref, w2rootT_ref, b2_ref,
                z_ref, rb_ref):
    agg = jnp.sum(aggp_ref[0], axis=0, keepdims=True)
    x = x_ref[0]
    ones = jnp.ones((1, x.shape[1]), jnp.float32)
    zeros = jnp.zeros((5, x.shape[1]), jnp.float32)
    a3 = jnp.concatenate([agg, x, ones, zeros], axis=0)
    hT = jax.nn.relu(
        jnp.dot(w3_ref[0], a3, preferred_element_type=jnp.float32))
    z_ref[...] = jnp.dot(w2relT_ref[0], hT,
                         preferred_element_type=jnp.float32)[None]
    rb_ref[...] = (jnp.dot(w2rootT_ref[0], hT,
                           preferred_element_type=jnp.float32)
                   + b2_ref[0][:, 0:1])[None]


def _dense(aggp, x2d, w3s, w2relTs, w2rootTs, b2ws):
    nb = NPAD // BLK
    return pl.pallas_call(
        _dense_body,
        grid=(2, nb),
        in_specs=[
            pl.BlockSpec((1, 16, BLK), lambda d, i: (1 - d, 0, i)),
            pl.BlockSpec((1, 1, BLK), lambda d, i: (d, 0, i)),
            pl.BlockSpec((1, H, 8), lambda d, i: (d, 0, 0)),
            pl.BlockSpec((1, O, H), lambda d, i: (d, 0, 0)),
            pl.BlockSpec((1, O, H), lambda d, i: (d, 0, 0)),
            pl.BlockSpec((1, O, 128), lambda d, i: (d, 0, 0)),
        ],
        out_specs=[pl.BlockSpec((1, O, BLK), lambda d, i: (d, 0, i)),
                   pl.BlockSpec((1, O, BLK), lambda d, i: (d, 0, i))],
        out_shape=[jax.ShapeDtypeStruct((2, O, NPAD), jnp.float32),
                   jax.ShapeDtypeStruct((2, O, NPAD), jnp.float32)],
    )(aggp, x2d, w3s, w2relTs, w2rootTs, b2ws)


def _pack(edge_index):
    pk = jnp.bitwise_or(edge_index[0],
                        jnp.left_shift(edge_index[1], 16)).astype(jnp.int32)
    pad = jnp.full((EPAD - E,), N | (N << 16), dtype=jnp.int32)
    return jnp.concatenate([pk, pad])


def _w3(w_rel, w_root, b):
    w = jnp.concatenate([w_rel, w_root, b[None, :],
                         jnp.zeros((5, H), jnp.float32)], axis=0)
    return w.T


def kernel(x_user, x_item, edge_index_u2i, edge_index_i2u,
           W1_rel_u2i, b1_rel_u2i, W1_root_u2i,
           W1_rel_i2u, b1_rel_i2u, W1_root_i2u,
           W2_rel_u2i, b2_rel_u2i, W2_root_u2i,
           W2_rel_i2u, b2_rel_i2u, W2_root_i2u):
    xu = jnp.pad(x_user[:, 0], (0, NPAD - N))
    xi = jnp.pad(x_item[:, 0], (0, NPAD - N))
    x2 = jnp.concatenate([xu, xi])
    pk2 = jnp.concatenate([_pack(edge_index_u2i), _pack(edge_index_i2u)])

    aggp = _sc_layer1(x2, pk2).reshape(2, 16, NPAD)

    x2d = x2.reshape(2, 1, NPAD)
    w3s = jnp.stack([_w3(W1_rel_i2u, W1_root_i2u, b1_rel_i2u),
                     _w3(W1_rel_u2i, W1_root_u2i, b1_rel_u2i)])
    w2relTs = jnp.stack([W2_rel_u2i.T, W2_rel_i2u.T])
    w2rootTs = jnp.stack([W2_root_i2u.T, W2_root_u2i.T])
    b2ws = jnp.stack([jnp.broadcast_to(b2_rel_i2u[:, None], (O, 128)),
                      jnp.broadcast_to(b2_rel_u2i[:, None], (O, 128))])
    zs, rbs = _dense(aggp, x2d, w3s, w2relTs, w2rootTs, b2ws)

    out2 = _sc_layer2(zs.reshape(-1), pk2, rbs.reshape(-1))
    out2 = out2.reshape(2, O, NPAD)
    out_item = out2[0].T[:N]
    out_user = out2[1].T[:N]
    return (out_user, out_item)
